# Initial kernel scaffold; baseline (speedup 1.0000x reference)
import jax
import jax.numpy as jnp
from jax.experimental import pallas as pl


def kernel(x, edge_index, W_vel, b_vel, W_res, b_res, W_lap, b_lap):
    raise NotImplementedError("write your pallas kernel here")


# speedup vs baseline: 4.1071x; 4.1071x over previous
import functools

import jax
import jax.numpy as jnp
from jax import lax
from jax.experimental import pallas as pl
from jax.experimental.pallas import tpu as pltpu
from jax.experimental.pallas import tpu_sc as plsc

D = 128
EPS = 0.01
ITERS = 2

NCORES = 2
NSUB = 16
NTILES = NCORES * NSUB
CHUNK = 128
LANES = 16

N_NODES = 10000
N_EDGES = 320000
NPAD = 10240
ROWS_PER_SUB = NPAD // NSUB
SB = 16
NSB = 5
NCHUNK = SB * NSB
EPAD = NTILES * NCHUNK * CHUNK
BN = 1024


def _vel_body(x_ref, w_ref, b_ref, o_ref):
    o_ref[...] = (
        jnp.dot(x_ref[...], w_ref[...], preferred_element_type=jnp.float32)
        + b_ref[...][None, :]
    )


def _prep_body(x_ref, w_ref, b_ref, w1_ref, w2_ref, feat_ref, p_ref, q_ref):
    xb = x_ref[...]
    feat_ref[...] = (
        jnp.dot(xb, w_ref[...], preferred_element_type=jnp.float32)
        + b_ref[...][None, :]
    )
    p_ref[...] = jnp.sum(xb * w1_ref[...][None, :], axis=1, keepdims=True)
    q_ref[...] = jnp.sum(xb * w2_ref[...][None, :], axis=1, keepdims=True)


def _update_body(x_ref, v_ref, feat_ref, a0_ref, a1_ref, d0_ref, d1_ref,
                 xo_ref, vo_ref):
    deg = d0_ref[...] + d1_ref[...]
    conv = deg * feat_ref[...] - a0_ref[...] - a1_ref[...]
    vn = v_ref[...] - EPS * conv
    vo_ref[...] = vn
    xo_ref[...] = x_ref[...] + EPS * vn


def _tc_vel(xp, w_t, b):
    grid = (NPAD // BN,)
    return pl.pallas_call(
        _vel_body,
        grid=grid,
        in_specs=[
            pl.BlockSpec((BN, D), lambda i: (i, 0)),
            pl.BlockSpec((D, D), lambda i: (0, 0)),
            pl.BlockSpec((D,), lambda i: (0,)),
        ],
        out_specs=pl.BlockSpec((BN, D), lambda i: (i, 0)),
        out_shape=jax.ShapeDtypeStruct((NPAD, D), jnp.float32),
    )(xp, w_t, b)


def _tc_prep(xp, w_t, b, w1, w2):
    grid = (NPAD // BN,)
    return pl.pallas_call(
        _prep_body,
        grid=grid,
        in_specs=[
            pl.BlockSpec((BN, D), lambda i: (i, 0)),
            pl.BlockSpec((D, D), lambda i: (0, 0)),
            pl.BlockSpec((D,), lambda i: (0,)),
            pl.BlockSpec((D,), lambda i: (0,)),
            pl.BlockSpec((D,), lambda i: (0,)),
        ],
        out_specs=[
            pl.BlockSpec((BN, D), lambda i: (i, 0)),
            pl.BlockSpec((BN, 1), lambda i: (i, 0)),
            pl.BlockSpec((BN, 1), lambda i: (i, 0)),
        ],
        out_shape=[
            jax.ShapeDtypeStruct((NPAD, D), jnp.float32),
            jax.ShapeDtypeStruct((NPAD, 1), jnp.float32),
            jax.ShapeDtypeStruct((NPAD, 1), jnp.float32),
        ],
    )(xp, w_t, b, w1, w2)


def _tc_update(xp, v, feat, adj0, adj1, deg0, deg1):
    grid = (NPAD // BN,)
    row = pl.BlockSpec((BN, D), lambda i: (i, 0))
    col = pl.BlockSpec((BN, 1), lambda i: (i, 0))
    return pl.pallas_call(
        _update_body,
        grid=grid,
        in_specs=[row, row, row, row, row, col, col],
        out_specs=[row, row],
        out_shape=[
            jax.ShapeDtypeStruct((NPAD, D), jnp.float32),
            jax.ShapeDtypeStruct((NPAD, D), jnp.float32),
        ],
    )(xp, v, feat, adj0, adj1, deg0, deg1)


_MESH = plsc.VectorSubcoreMesh(
    core_axis_name="c", subcore_axis_name="s",
    num_cores=NCORES, num_subcores=NSUB,
)


@functools.partial(
    pl.kernel,
    out_type=(
        jax.ShapeDtypeStruct((NCORES, NPAD, D), jnp.float32),
        jax.ShapeDtypeStruct((NCORES, NPAD), jnp.float32),
    ),
    mesh=_MESH,
    compiler_params=pltpu.CompilerParams(needs_layout_passes=False),
    scratch_types=[
        pltpu.VMEM((NPAD,), jnp.float32),
        pltpu.VMEM((NPAD,), jnp.float32),
        pltpu.VMEM((SB, CHUNK), jnp.int32),
        pltpu.VMEM((SB, CHUNK), jnp.int32),
        pltpu.VMEM((SB, CHUNK), jnp.float32),
        pltpu.VMEM((LANES,), jnp.float32),
        pltpu.VMEM((CHUNK, D), jnp.float32),
        pltpu.VMEM_SHARED((NPAD, D), jnp.float32),
        pltpu.VMEM_SHARED((NPAD,), jnp.float32),
        pltpu.SemaphoreType.DMA,
    ],
)
def _sc_edges(feat_hbm, p_hbm, q_hbm, src_hbm, dst_hbm, b_hbm,
              adj_out, deg_out,
              p_v, q_v, src_v, dst_v, r_v, b_v, rows_v, acc, dega, sem):
    c = lax.axis_index("c")
    s = lax.axis_index("s")
    wid = c * NSUB + s
    base = s * ROWS_PER_SUB

    zero16 = jnp.zeros((LANES,), jnp.float32)

    def zrow(e, carry):
        for d8 in range(D // LANES):
            rows_v[e, pl.ds(d8 * LANES, LANES)] = zero16
        return carry

    lax.fori_loop(0, CHUNK, zrow, 0)

    for k in range(ROWS_PER_SUB // CHUNK):
        pltpu.sync_copy(rows_v, acc.at[pl.ds(base + k * CHUNK, CHUNK)])
        pltpu.sync_copy(rows_v.at[0],
                        dega.at[pl.ds(base + k * CHUNK, CHUNK)])

    pltpu.sync_copy(p_hbm, p_v)
    pltpu.sync_copy(q_hbm, q_v)
    pltpu.sync_copy(b_hbm, b_v)
    plsc.subcore_barrier()

    bb = b_v[...]

    def sblock(b, carry):
        pltpu.sync_copy(src_hbm.at[wid, pl.ds(b * SB, SB)], src_v)
        pltpu.sync_copy(dst_hbm.at[wid, pl.ds(b * SB, SB)], dst_v)

        def rstep(j, carry1):
            for k8 in range(CHUNK // LANES):
                sl = pl.ds(k8 * LANES, LANES)
                s16 = src_v[j, sl]
                d16 = dst_v[j, sl]
                ps = plsc.load_gather(p_v, [s16])
                qd = plsc.load_gather(q_v, [d16])
                rr = jnp.maximum(ps + qd + bb, 0.0)
                rr = jnp.where(s16 != d16, rr, 0.0)
                r_v[j, sl] = rr
            return carry1

        lax.fori_loop(0, SB, rstep, 0)

        def mstep(j, carry1):
            pltpu.async_copy(feat_hbm.at[src_v.at[j]], rows_v, sem).wait()

            def escale(e, carry2):
                je = jnp.full((LANES,), j, jnp.int32)
                ee = jnp.full((LANES,), e, jnp.int32)
                rv = plsc.load_gather(r_v, [je, ee])
                for d8 in range(D // LANES):
                    sl = pl.ds(d8 * LANES, LANES)
                    rows_v[e, sl] = rows_v[e, sl] * rv
                return carry2

            lax.fori_loop(0, CHUNK, escale, 0)
            pltpu.sync_copy(rows_v, acc.at[dst_v.at[j]], add=True)
            pltpu.sync_copy(r_v.at[j], dega.at[src_v.at[j]], add=True)
            return carry1

        lax.fori_loop(0, SB, mstep, 0)
        return carry

    lax.fori_loop(0, NSB, sblock, 0)

    plsc.subcore_barrier()

    pltpu.sync_copy(acc.at[pl.ds(base, ROWS_PER_SUB)],
                    adj_out.at[c].at[pl.ds(base, ROWS_PER_SUB)])
    pltpu.sync_copy(dega.at[pl.ds(base, ROWS_PER_SUB)],
                    deg_out.at[c].at[pl.ds(base, ROWS_PER_SUB)])


def kernel(x, edge_index, W_vel, b_vel, W_res, b_res, W_lap, b_lap):
    n, d = x.shape
    e = edge_index.shape[1]

    xp = jnp.zeros((NPAD, D), jnp.float32).at[:n].set(x)
    srcp = jnp.zeros((EPAD,), jnp.int32).at[:e].set(edge_index[0])
    dstp = jnp.zeros((EPAD,), jnp.int32).at[:e].set(edge_index[1])
    srcp = srcp.reshape(NTILES, NCHUNK, CHUNK)
    dstp = dstp.reshape(NTILES, NCHUNK, CHUNK)
    w1 = W_res[0, :d]
    w2 = W_res[0, d:]
    b16 = jnp.full((LANES,), b_res[0], jnp.float32)

    v = _tc_vel(xp, W_vel.T, b_vel)
    for _ in range(ITERS):
        feat, p, q = _tc_prep(xp, W_lap.T, b_lap, w1, w2)
        adj, deg = _sc_edges(feat, p.reshape(NPAD), q.reshape(NPAD),
                             srcp, dstp, b16)
        xp, v = _tc_update(xp, v, feat, adj[0], adj[1],
                           deg[0].reshape(NPAD, 1), deg[1].reshape(NPAD, 1))
    return xp[:n]


# speedup vs baseline: 9.1236x; 2.2214x over previous
import functools

import jax
import jax.numpy as jnp
from jax import lax
from jax.experimental import pallas as pl
from jax.experimental.pallas import tpu as pltpu
from jax.experimental.pallas import tpu_sc as plsc

D = 128
EPS = 0.01
ITERS = 2

NCORES = 2
NSUB = 16
NTILES = NCORES * NSUB
CHUNK = 128
LANES = 16

N_NODES = 10000
N_EDGES = 320000
NPAD = 10240
ROWS_PER_SUB = NPAD // NSUB
SB = 16
NSB = 5
NCHUNK = SB * NSB
EPAD = NTILES * NCHUNK * CHUNK
BN = 1024


def _vel_body(x_ref, w_ref, b_ref, o_ref):
    o_ref[...] = (
        jnp.dot(x_ref[...], w_ref[...], preferred_element_type=jnp.float32)
        + b_ref[...][None, :]
    )


def _prep_body(x_ref, w_ref, b_ref, w1_ref, w2_ref, feat_ref, p_ref, q_ref):
    xb = x_ref[...]
    feat_ref[...] = (
        jnp.dot(xb, w_ref[...], preferred_element_type=jnp.float32)
        + b_ref[...][None, :]
    )
    p_ref[...] = jnp.sum(xb * w1_ref[...][None, :], axis=1, keepdims=True)
    q_ref[...] = jnp.sum(xb * w2_ref[...][None, :], axis=1, keepdims=True)


def _update_body(x_ref, v_ref, feat_ref, a0_ref, a1_ref, d0_ref, d1_ref,
                 xo_ref, vo_ref):
    deg = d0_ref[...] + d1_ref[...]
    conv = deg * feat_ref[...] - a0_ref[...] - a1_ref[...]
    vn = v_ref[...] - EPS * conv
    vo_ref[...] = vn
    xo_ref[...] = x_ref[...] + EPS * vn


def _tc_vel(xp, w_t, b):
    grid = (NPAD // BN,)
    return pl.pallas_call(
        _vel_body,
        grid=grid,
        in_specs=[
            pl.BlockSpec((BN, D), lambda i: (i, 0)),
            pl.BlockSpec((D, D), lambda i: (0, 0)),
            pl.BlockSpec((D,), lambda i: (0,)),
        ],
        out_specs=pl.BlockSpec((BN, D), lambda i: (i, 0)),
        out_shape=jax.ShapeDtypeStruct((NPAD, D), jnp.float32),
    )(xp, w_t, b)


def _tc_prep(xp, w_t, b, w1, w2):
    grid = (NPAD // BN,)
    return pl.pallas_call(
        _prep_body,
        grid=grid,
        in_specs=[
            pl.BlockSpec((BN, D), lambda i: (i, 0)),
            pl.BlockSpec((D, D), lambda i: (0, 0)),
            pl.BlockSpec((D,), lambda i: (0,)),
            pl.BlockSpec((D,), lambda i: (0,)),
            pl.BlockSpec((D,), lambda i: (0,)),
        ],
        out_specs=[
            pl.BlockSpec((BN, D), lambda i: (i, 0)),
            pl.BlockSpec((BN, 1), lambda i: (i, 0)),
            pl.BlockSpec((BN, 1), lambda i: (i, 0)),
        ],
        out_shape=[
            jax.ShapeDtypeStruct((NPAD, D), jnp.float32),
            jax.ShapeDtypeStruct((NPAD, 1), jnp.float32),
            jax.ShapeDtypeStruct((NPAD, 1), jnp.float32),
        ],
    )(xp, w_t, b, w1, w2)


def _tc_update(xp, v, feat, adj0, adj1, deg0, deg1):
    grid = (NPAD // BN,)
    row = pl.BlockSpec((BN, D), lambda i: (i, 0))
    col = pl.BlockSpec((BN, 1), lambda i: (i, 0))
    return pl.pallas_call(
        _update_body,
        grid=grid,
        in_specs=[row, row, row, row, row, col, col],
        out_specs=[row, row],
        out_shape=[
            jax.ShapeDtypeStruct((NPAD, D), jnp.float32),
            jax.ShapeDtypeStruct((NPAD, D), jnp.float32),
        ],
    )(xp, v, feat, adj0, adj1, deg0, deg1)


_MESH = plsc.VectorSubcoreMesh(
    core_axis_name="c", subcore_axis_name="s",
    num_cores=NCORES, num_subcores=NSUB,
)


@functools.partial(
    pl.kernel,
    out_type=(
        jax.ShapeDtypeStruct((NCORES, NPAD, D), jnp.float32),
        jax.ShapeDtypeStruct((NCORES, NPAD), jnp.float32),
    ),
    mesh=_MESH,
    compiler_params=pltpu.CompilerParams(needs_layout_passes=False),
    scratch_types=[
        pltpu.VMEM((NPAD,), jnp.float32),
        pltpu.VMEM((NPAD,), jnp.float32),
        pltpu.VMEM((SB, CHUNK), jnp.int32),
        pltpu.VMEM((SB, CHUNK), jnp.int32),
        pltpu.VMEM((SB, CHUNK), jnp.float32),
        pltpu.VMEM((LANES,), jnp.float32),
        pltpu.VMEM((CHUNK, D), jnp.float32),
        pltpu.VMEM_SHARED((NPAD, D), jnp.float32),
        pltpu.VMEM_SHARED((NPAD,), jnp.float32),
        pltpu.SemaphoreType.DMA,
    ],
)
def _sc_edges(feat_hbm, p_hbm, q_hbm, src_hbm, dst_hbm, b_hbm,
              adj_out, deg_out,
              p_v, q_v, src_v, dst_v, r_v, b_v, rows_v, acc, dega, sem):
    c = lax.axis_index("c")
    s = lax.axis_index("s")
    wid = c * NSUB + s
    base = s * ROWS_PER_SUB

    zero16 = jnp.zeros((LANES,), jnp.float32)

    def zrow(e, carry):
        for d8 in range(D // LANES):
            rows_v[e, pl.ds(d8 * LANES, LANES)] = zero16
        return carry

    lax.fori_loop(0, CHUNK, zrow, 0)

    for k in range(ROWS_PER_SUB // CHUNK):
        pltpu.sync_copy(rows_v, acc.at[pl.ds(base + k * CHUNK, CHUNK)])
        pltpu.sync_copy(rows_v.at[0],
                        dega.at[pl.ds(base + k * CHUNK, CHUNK)])

    pltpu.sync_copy(p_hbm, p_v)
    pltpu.sync_copy(q_hbm, q_v)
    pltpu.sync_copy(b_hbm, b_v)
    plsc.subcore_barrier()

    bb = b_v[...]

    def sblock(b, carry):
        pltpu.sync_copy(src_hbm.at[wid, pl.ds(b * SB, SB)], src_v)
        pltpu.sync_copy(dst_hbm.at[wid, pl.ds(b * SB, SB)], dst_v)

        def rstep(j, carry1):
            for k8 in range(CHUNK // LANES):
                sl = pl.ds(k8 * LANES, LANES)
                s16 = src_v[j, sl]
                d16 = dst_v[j, sl]
                ps = plsc.load_gather(p_v, [s16])
                qd = plsc.load_gather(q_v, [d16])
                rr = jnp.maximum(ps + qd + bb, 0.0)
                rr = jnp.where(s16 != d16, rr, 0.0)
                r_v[j, sl] = rr
            return carry1

        lax.fori_loop(0, SB, rstep, 0)

        def mstep(j, carry1):
            gstart = (wid * NCHUNK + b * SB + j) * CHUNK

            @pl.when(gstart < N_EDGES)
            def _do_chunk():
                pltpu.async_copy(feat_hbm.at[src_v.at[j]], rows_v, sem).wait()

                def escale(e, carry2):
                    je = jnp.full((LANES,), j, jnp.int32)
                    ee = jnp.full((LANES,), e, jnp.int32)
                    rv = plsc.load_gather(r_v, [je, ee])
                    for d8 in range(D // LANES):
                        sl = pl.ds(d8 * LANES, LANES)
                        rows_v[e, sl] = rows_v[e, sl] * rv
                    return carry2

                lax.fori_loop(0, CHUNK, escale, 0)
                pltpu.sync_copy(rows_v, acc.at[dst_v.at[j]], add=True)
                pltpu.sync_copy(r_v.at[j], dega.at[src_v.at[j]], add=True)

            return carry1

        lax.fori_loop(0, SB, mstep, 0)
        return carry

    lax.fori_loop(0, NSB, sblock, 0)

    plsc.subcore_barrier()

    pltpu.sync_copy(acc.at[pl.ds(base, ROWS_PER_SUB)],
                    adj_out.at[c].at[pl.ds(base, ROWS_PER_SUB)])
    pltpu.sync_copy(dega.at[pl.ds(base, ROWS_PER_SUB)],
                    deg_out.at[c].at[pl.ds(base, ROWS_PER_SUB)])


def kernel(x, edge_index, W_vel, b_vel, W_res, b_res, W_lap, b_lap):
    n, d = x.shape
    e = edge_index.shape[1]

    xp = jnp.zeros((NPAD, D), jnp.float32).at[:n].set(x)
    srcp = jnp.zeros((EPAD,), jnp.int32).at[:e].set(edge_index[0])
    dstp = jnp.zeros((EPAD,), jnp.int32).at[:e].set(edge_index[1])
    srcp = srcp.reshape(NTILES, NCHUNK, CHUNK)
    dstp = dstp.reshape(NTILES, NCHUNK, CHUNK)
    w1 = W_res[0, :d]
    w2 = W_res[0, d:]
    b16 = jnp.full((LANES,), b_res[0], jnp.float32)

    v = _tc_vel(xp, W_vel.T, b_vel)
    for _ in range(ITERS):
        feat, p, q = _tc_prep(xp, W_lap.T, b_lap, w1, w2)
        adj, deg = _sc_edges(feat, p.reshape(NPAD), q.reshape(NPAD),
                             srcp, dstp, b16)
        xp, v = _tc_update(xp, v, feat, adj[0], adj[1],
                           deg[0].reshape(NPAD, 1), deg[1].reshape(NPAD, 1))
    return xp[:n]


# speedup vs baseline: 10.2759x; 1.1263x over previous
import functools

import jax
import jax.numpy as jnp
from jax import lax
from jax.experimental import pallas as pl
from jax.experimental.pallas import tpu as pltpu
from jax.experimental.pallas import tpu_sc as plsc

D = 128
EPS = 0.01
ITERS = 2

NCORES = 2
NSUB = 16
NTILES = NCORES * NSUB
CHUNK = 128
LANES = 16

N_NODES = 10000
N_EDGES = 320000
NPAD = 10240
ROWS_PER_SUB = NPAD // NSUB
SB = 16
NSB = 5
NCHUNK = SB * NSB
EPAD = NTILES * NCHUNK * CHUNK
BN = 1024


def _vel_body(x_ref, w_ref, b_ref, o_ref):
    o_ref[...] = (
        jnp.dot(x_ref[...], w_ref[...], preferred_element_type=jnp.float32)
        + b_ref[...][None, :]
    )


def _prep_body(x_ref, w_ref, b_ref, w1_ref, w2_ref, feat_ref, p_ref, q_ref):
    xb = x_ref[...]
    feat_ref[...] = (
        jnp.dot(xb, w_ref[...], preferred_element_type=jnp.float32)
        + b_ref[...][None, :]
    )
    p_ref[...] = jnp.sum(xb * w1_ref[...][None, :], axis=1, keepdims=True)
    q_ref[...] = jnp.sum(xb * w2_ref[...][None, :], axis=1, keepdims=True)


def _update_body(x_ref, v_ref, feat_ref, a0_ref, a1_ref, d0_ref, d1_ref,
                 xo_ref, vo_ref):
    deg = d0_ref[...] + d1_ref[...]
    conv = deg * feat_ref[...] - a0_ref[...] - a1_ref[...]
    vn = v_ref[...] - EPS * conv
    vo_ref[...] = vn
    xo_ref[...] = x_ref[...] + EPS * vn


def _tc_vel(xp, w_t, b):
    grid = (NPAD // BN,)
    return pl.pallas_call(
        _vel_body,
        grid=grid,
        in_specs=[
            pl.BlockSpec((BN, D), lambda i: (i, 0)),
            pl.BlockSpec((D, D), lambda i: (0, 0)),
            pl.BlockSpec((D,), lambda i: (0,)),
        ],
        out_specs=pl.BlockSpec((BN, D), lambda i: (i, 0)),
        out_shape=jax.ShapeDtypeStruct((NPAD, D), jnp.float32),
    )(xp, w_t, b)


def _tc_prep(xp, w_t, b, w1, w2):
    grid = (NPAD // BN,)
    return pl.pallas_call(
        _prep_body,
        grid=grid,
        in_specs=[
            pl.BlockSpec((BN, D), lambda i: (i, 0)),
            pl.BlockSpec((D, D), lambda i: (0, 0)),
            pl.BlockSpec((D,), lambda i: (0,)),
            pl.BlockSpec((D,), lambda i: (0,)),
            pl.BlockSpec((D,), lambda i: (0,)),
        ],
        out_specs=[
            pl.BlockSpec((BN, D), lambda i: (i, 0)),
            pl.BlockSpec((BN, 1), lambda i: (i, 0)),
            pl.BlockSpec((BN, 1), lambda i: (i, 0)),
        ],
        out_shape=[
            jax.ShapeDtypeStruct((NPAD, D), jnp.float32),
            jax.ShapeDtypeStruct((NPAD, 1), jnp.float32),
            jax.ShapeDtypeStruct((NPAD, 1), jnp.float32),
        ],
    )(xp, w_t, b, w1, w2)


def _tc_update(xp, v, feat, adj0, adj1, deg0, deg1):
    grid = (NPAD // BN,)
    row = pl.BlockSpec((BN, D), lambda i: (i, 0))
    col = pl.BlockSpec((BN, 1), lambda i: (i, 0))
    return pl.pallas_call(
        _update_body,
        grid=grid,
        in_specs=[row, row, row, row, row, col, col],
        out_specs=[row, row],
        out_shape=[
            jax.ShapeDtypeStruct((NPAD, D), jnp.float32),
            jax.ShapeDtypeStruct((NPAD, D), jnp.float32),
        ],
    )(xp, v, feat, adj0, adj1, deg0, deg1)


_MESH = plsc.VectorSubcoreMesh(
    core_axis_name="c", subcore_axis_name="s",
    num_cores=NCORES, num_subcores=NSUB,
)


@functools.partial(
    pl.kernel,
    out_type=(
        jax.ShapeDtypeStruct((NCORES, NPAD, D), jnp.float32),
        jax.ShapeDtypeStruct((NCORES, NPAD), jnp.float32),
    ),
    mesh=_MESH,
    compiler_params=pltpu.CompilerParams(needs_layout_passes=False),
    scratch_types=[
        pltpu.VMEM((SB, CHUNK), jnp.int32),
        pltpu.VMEM((SB, CHUNK), jnp.int32),
        pltpu.VMEM((SB, CHUNK), jnp.float32),
        pltpu.VMEM((SB, CHUNK), jnp.float32),
        pltpu.VMEM((SB, CHUNK), jnp.float32),
        pltpu.VMEM((LANES,), jnp.float32),
        pltpu.VMEM((CHUNK, D), jnp.float32),
        pltpu.VMEM((CHUNK, D), jnp.float32),
        pltpu.VMEM_SHARED((NPAD, D), jnp.float32),
        pltpu.VMEM_SHARED((NPAD,), jnp.float32),
        pltpu.SemaphoreType.DMA,
        pltpu.SemaphoreType.DMA,
    ],
)
def _sc_edges(feat_hbm, p_hbm, q_hbm, src_hbm, dst_hbm, b_hbm,
              adj_out, deg_out,
              src_v, dst_v, r_v, ps_v, qd_v, b_v, rows0, rows1,
              acc, dega, sem_g, sem_pq):
    c = lax.axis_index("c")
    s = lax.axis_index("s")
    wid = c * NSUB + s
    base = s * ROWS_PER_SUB

    zero16 = jnp.zeros((LANES,), jnp.float32)

    def zrow(e, carry):
        for d8 in range(D // LANES):
            rows0[e, pl.ds(d8 * LANES, LANES)] = zero16
        return carry

    lax.fori_loop(0, CHUNK, zrow, 0)

    for k in range(ROWS_PER_SUB // CHUNK):
        pltpu.sync_copy(rows0, acc.at[pl.ds(base + k * CHUNK, CHUNK)])
        pltpu.sync_copy(rows0.at[0],
                        dega.at[pl.ds(base + k * CHUNK, CHUNK)])

    pltpu.sync_copy(b_hbm, b_v)
    plsc.subcore_barrier()

    bb = b_v[...]
    bufs = (rows0, rows1)

    def start_gather(j, buf):
        pltpu.async_copy(feat_hbm.at[src_v.at[j]], buf, sem_g)

    def wait_gather(buf):
        pltpu.make_async_copy(feat_hbm.at[pl.ds(0, CHUNK)], buf, sem_g).wait()

    def sblock(b, carry):
        pltpu.sync_copy(src_hbm.at[wid, pl.ds(b * SB, SB)], src_v)
        pltpu.sync_copy(dst_hbm.at[wid, pl.ds(b * SB, SB)], dst_v)

        def active(j):
            return (wid * NCHUNK + b * SB + j) * CHUNK < N_EDGES

        pq = [pltpu.async_copy(p_hbm.at[src_v.at[js]], ps_v.at[js], sem_pq)
              for js in range(SB)]
        pq += [pltpu.async_copy(q_hbm.at[dst_v.at[js]], qd_v.at[js], sem_pq)
               for js in range(SB)]

        @pl.when(active(0))
        def _g0():
            start_gather(0, rows0)

        for dsc in pq:
            dsc.wait()

        def rstep(j, carry1):
            for k8 in range(CHUNK // LANES):
                sl = pl.ds(k8 * LANES, LANES)
                s16 = src_v[j, sl]
                d16 = dst_v[j, sl]
                rr = jnp.maximum(ps_v[j, sl] + qd_v[j, sl] + bb, 0.0)
                rr = jnp.where(s16 != d16, rr, 0.0)
                r_v[j, sl] = rr
            return carry1

        lax.fori_loop(0, SB, rstep, 0)

        def mpair(jj, carry1):
            for par in range(2):
                j = 2 * jj + par
                buf = bufs[par]
                other = bufs[1 - par]

                @pl.when(active(j))
                def _do_chunk():
                    wait_gather(buf)

                    if par == 0:
                        nxt = j + 1

                        @pl.when(active(nxt))
                        def _gn():
                            start_gather(nxt, other)
                    else:
                        nxt = j + 1

                        @pl.when((nxt < SB) & active(nxt))
                        def _gn():
                            start_gather(nxt, other)

                    def escale(e, carry2):
                        je = jnp.full((LANES,), j, jnp.int32)
                        ee = jnp.full((LANES,), e, jnp.int32)
                        rv = plsc.load_gather(r_v, [je, ee])
                        for d8 in range(D // LANES):
                            sl = pl.ds(d8 * LANES, LANES)
                            buf[e, sl] = buf[e, sl] * rv
                        return carry2

                    lax.fori_loop(0, CHUNK, escale, 0)
                    pltpu.sync_copy(buf, acc.at[dst_v.at[j]], add=True)
                    pltpu.sync_copy(r_v.at[j], dega.at[src_v.at[j]],
                                    add=True)

            return carry1

        lax.fori_loop(0, SB // 2, mpair, 0)
        return carry

    lax.fori_loop(0, NSB, sblock, 0)

    plsc.subcore_barrier()

    pltpu.sync_copy(acc.at[pl.ds(base, ROWS_PER_SUB)],
                    adj_out.at[c].at[pl.ds(base, ROWS_PER_SUB)])
    pltpu.sync_copy(dega.at[pl.ds(base, ROWS_PER_SUB)],
                    deg_out.at[c].at[pl.ds(base, ROWS_PER_SUB)])


def kernel(x, edge_index, W_vel, b_vel, W_res, b_res, W_lap, b_lap):
    n, d = x.shape
    e = edge_index.shape[1]

    xp = jnp.zeros((NPAD, D), jnp.float32).at[:n].set(x)
    srcp = jnp.zeros((EPAD,), jnp.int32).at[:e].set(edge_index[0])
    dstp = jnp.zeros((EPAD,), jnp.int32).at[:e].set(edge_index[1])
    srcp = srcp.reshape(NTILES, NCHUNK, CHUNK)
    dstp = dstp.reshape(NTILES, NCHUNK, CHUNK)
    w1 = W_res[0, :d]
    w2 = W_res[0, d:]
    b16 = jnp.full((LANES,), b_res[0], jnp.float32)

    v = _tc_vel(xp, W_vel.T, b_vel)
    for _ in range(ITERS):
        feat, p, q = _tc_prep(xp, W_lap.T, b_lap, w1, w2)
        adj, deg = _sc_edges(feat, p.reshape(NPAD), q.reshape(NPAD),
                             srcp, dstp, b16)
        xp, v = _tc_update(xp, v, feat, adj[0], adj[1],
                           deg[0].reshape(NPAD, 1), deg[1].reshape(NPAD, 1))
    return xp[:n]


# speedup vs baseline: 11.3025x; 1.0999x over previous
import functools

import jax
import jax.numpy as jnp
from jax import lax
from jax.experimental import pallas as pl
from jax.experimental.pallas import tpu as pltpu
from jax.experimental.pallas import tpu_sc as plsc

D = 128
EPS = 0.01
ITERS = 2

NCORES = 2
NSUB = 16
NTILES = NCORES * NSUB
CHUNK = 128
LANES = 16

N_NODES = 10000
N_EDGES = 320000
NPAD = 10240
ROWS_PER_SUB = NPAD // NSUB
SB = 16
NSB = 5
NCHUNK = SB * NSB
EPAD = NTILES * NCHUNK * CHUNK
BN = 1024


def _vel_body(x_ref, w_ref, b_ref, o_ref):
    o_ref[...] = (
        jnp.dot(x_ref[...], w_ref[...], preferred_element_type=jnp.float32)
        + b_ref[...][None, :]
    )


def _prep_body(x_ref, w_ref, b_ref, w1_ref, w2_ref, feat_ref, p_ref, q_ref):
    xb = x_ref[...]
    feat_ref[...] = (
        jnp.dot(xb, w_ref[...], preferred_element_type=jnp.float32)
        + b_ref[...][None, :]
    )
    p_ref[...] = jnp.sum(xb * w1_ref[...][None, :], axis=1, keepdims=True)
    q_ref[...] = jnp.sum(xb * w2_ref[...][None, :], axis=1, keepdims=True)


def _update_body(x_ref, v_ref, feat_ref, a0_ref, a1_ref, d0_ref, d1_ref,
                 xo_ref, vo_ref):
    deg = d0_ref[...] + d1_ref[...]
    conv = deg * feat_ref[...] - a0_ref[...] - a1_ref[...]
    vn = v_ref[...] - EPS * conv
    vo_ref[...] = vn
    xo_ref[...] = x_ref[...] + EPS * vn


def _tc_vel(xp, w_t, b):
    grid = (NPAD // BN,)
    return pl.pallas_call(
        _vel_body,
        grid=grid,
        in_specs=[
            pl.BlockSpec((BN, D), lambda i: (i, 0)),
            pl.BlockSpec((D, D), lambda i: (0, 0)),
            pl.BlockSpec((D,), lambda i: (0,)),
        ],
        out_specs=pl.BlockSpec((BN, D), lambda i: (i, 0)),
        out_shape=jax.ShapeDtypeStruct((NPAD, D), jnp.float32),
    )(xp, w_t, b)


def _tc_prep(xp, w_t, b, w1, w2):
    grid = (NPAD // BN,)
    return pl.pallas_call(
        _prep_body,
        grid=grid,
        in_specs=[
            pl.BlockSpec((BN, D), lambda i: (i, 0)),
            pl.BlockSpec((D, D), lambda i: (0, 0)),
            pl.BlockSpec((D,), lambda i: (0,)),
            pl.BlockSpec((D,), lambda i: (0,)),
            pl.BlockSpec((D,), lambda i: (0,)),
        ],
        out_specs=[
            pl.BlockSpec((BN, D), lambda i: (i, 0)),
            pl.BlockSpec((BN, 1), lambda i: (i, 0)),
            pl.BlockSpec((BN, 1), lambda i: (i, 0)),
        ],
        out_shape=[
            jax.ShapeDtypeStruct((NPAD, D), jnp.float32),
            jax.ShapeDtypeStruct((NPAD, 1), jnp.float32),
            jax.ShapeDtypeStruct((NPAD, 1), jnp.float32),
        ],
    )(xp, w_t, b, w1, w2)


def _tc_update(xp, v, feat, adj0, adj1, deg0, deg1):
    grid = (NPAD // BN,)
    row = pl.BlockSpec((BN, D), lambda i: (i, 0))
    col = pl.BlockSpec((BN, 1), lambda i: (i, 0))
    return pl.pallas_call(
        _update_body,
        grid=grid,
        in_specs=[row, row, row, row, row, col, col],
        out_specs=[row, row],
        out_shape=[
            jax.ShapeDtypeStruct((NPAD, D), jnp.float32),
            jax.ShapeDtypeStruct((NPAD, D), jnp.float32),
        ],
    )(xp, v, feat, adj0, adj1, deg0, deg1)


_MESH = plsc.VectorSubcoreMesh(
    core_axis_name="c", subcore_axis_name="s",
    num_cores=NCORES, num_subcores=NSUB,
)


@functools.partial(
    pl.kernel,
    out_type=(
        jax.ShapeDtypeStruct((NCORES, NPAD, D), jnp.float32),
        jax.ShapeDtypeStruct((NCORES, NPAD), jnp.float32),
    ),
    mesh=_MESH,
    compiler_params=pltpu.CompilerParams(needs_layout_passes=False),
    scratch_types=[
        pltpu.VMEM((SB, CHUNK), jnp.int32),
        pltpu.VMEM((SB, CHUNK), jnp.int32),
        pltpu.VMEM((SB, CHUNK), jnp.float32),
        pltpu.VMEM((SB, CHUNK), jnp.float32),
        pltpu.VMEM((SB, CHUNK), jnp.float32),
        pltpu.VMEM((LANES,), jnp.float32),
        pltpu.VMEM((CHUNK, D), jnp.float32),
        pltpu.VMEM((CHUNK, D), jnp.float32),
        pltpu.VMEM_SHARED((NPAD, D), jnp.float32),
        pltpu.VMEM_SHARED((NPAD,), jnp.float32),
        pltpu.SemaphoreType.DMA,
        pltpu.SemaphoreType.DMA,
        pltpu.SemaphoreType.DMA,
        pltpu.SemaphoreType.DMA,
    ],
)
def _sc_edges(feat_hbm, p_hbm, q_hbm, src_hbm, dst_hbm, b_hbm,
              adj_out, deg_out,
              src_v, dst_v, r_v, ps_v, qd_v, b_v, rows0, rows1,
              acc, dega, sem_g, sem_pq, sem_s, sem_d):
    c = lax.axis_index("c")
    s = lax.axis_index("s")
    wid = c * NSUB + s
    base = s * ROWS_PER_SUB

    zero16 = jnp.zeros((LANES,), jnp.float32)

    def zrow(e, carry):
        for d8 in range(D // LANES):
            rows0[e, pl.ds(d8 * LANES, LANES)] = zero16
        return carry

    lax.fori_loop(0, CHUNK, zrow, 0)

    for k in range(ROWS_PER_SUB // CHUNK):
        pltpu.sync_copy(rows0, acc.at[pl.ds(base + k * CHUNK, CHUNK)])
        pltpu.sync_copy(rows0.at[0],
                        dega.at[pl.ds(base + k * CHUNK, CHUNK)])

    pltpu.sync_copy(b_hbm, b_v)
    plsc.subcore_barrier()

    bb = b_v[...]
    bufs = (rows0, rows1)

    def start_gather(j, buf):
        pltpu.async_copy(feat_hbm.at[src_v.at[j]], buf, sem_g)

    def wait_gather(buf):
        pltpu.make_async_copy(feat_hbm.at[pl.ds(0, CHUNK)], buf, sem_g).wait()

    def wait_scatter(buf):
        pltpu.make_async_copy(feat_hbm.at[pl.ds(0, CHUNK)], buf, sem_s).wait()

    def sblock(b, carry):
        pltpu.sync_copy(src_hbm.at[wid, pl.ds(b * SB, SB)], src_v)
        pltpu.sync_copy(dst_hbm.at[wid, pl.ds(b * SB, SB)], dst_v)

        def active(j):
            return (wid * NCHUNK + b * SB + j) * CHUNK < N_EDGES

        pq = [pltpu.async_copy(p_hbm.at[src_v.at[js]], ps_v.at[js], sem_pq)
              for js in range(SB)]
        pq += [pltpu.async_copy(q_hbm.at[dst_v.at[js]], qd_v.at[js], sem_pq)
               for js in range(SB)]

        @pl.when(active(0))
        def _g0():
            start_gather(0, rows0)

        for dsc in pq:
            dsc.wait()

        def rstep(j, carry1):
            for k8 in range(CHUNK // LANES):
                sl = pl.ds(k8 * LANES, LANES)
                s16 = src_v[j, sl]
                d16 = dst_v[j, sl]
                rr = jnp.maximum(ps_v[j, sl] + qd_v[j, sl] + bb, 0.0)
                rr = jnp.where(s16 != d16, rr, 0.0)
                r_v[j, sl] = rr
            return carry1

        lax.fori_loop(0, SB, rstep, 0)

        for js in range(SB):
            pltpu.async_copy(r_v.at[js], dega.at[src_v.at[js]], sem_d,
                             add=True)

        def mpair(jj, carry1):
            for par in range(2):
                j = 2 * jj + par
                buf = bufs[par]
                other = bufs[1 - par]

                @pl.when(active(j))
                def _do_chunk():
                    wait_gather(buf)

                    @pl.when(b * SB + j >= 1)
                    def _drain_prev():
                        wait_scatter(other)

                    nxt = j + 1
                    if par == 0:
                        @pl.when(active(nxt))
                        def _gn():
                            start_gather(nxt, other)
                    else:
                        @pl.when((nxt < SB) & active(nxt))
                        def _gn():
                            start_gather(nxt, other)

                    @plsc.parallel_loop(0, CHUNK, unroll=4)
                    def _escale(e):
                        je = jnp.full((LANES,), j, jnp.int32)
                        ee = jnp.full((LANES,), e, jnp.int32)
                        rv = plsc.load_gather(r_v, [je, ee])
                        for d8 in range(D // LANES):
                            sl = pl.ds(d8 * LANES, LANES)
                            buf[e, sl] = buf[e, sl] * rv

                    pltpu.async_copy(buf, acc.at[dst_v.at[j]], sem_s,
                                     add=True)

            return carry1

        lax.fori_loop(0, SB // 2, mpair, 0)

        pltpu.make_async_copy(feat_hbm.at[pl.ds(0, SB)], r_v, sem_d).wait()
        return carry

    lax.fori_loop(0, NSB, sblock, 0)

    wait_scatter(rows0)

    plsc.subcore_barrier()

    pltpu.sync_copy(acc.at[pl.ds(base, ROWS_PER_SUB)],
                    adj_out.at[c].at[pl.ds(base, ROWS_PER_SUB)])
    pltpu.sync_copy(dega.at[pl.ds(base, ROWS_PER_SUB)],
                    deg_out.at[c].at[pl.ds(base, ROWS_PER_SUB)])


def kernel(x, edge_index, W_vel, b_vel, W_res, b_res, W_lap, b_lap):
    n, d = x.shape
    e = edge_index.shape[1]

    xp = jnp.zeros((NPAD, D), jnp.float32).at[:n].set(x)
    srcp = jnp.zeros((EPAD,), jnp.int32).at[:e].set(edge_index[0])
    dstp = jnp.zeros((EPAD,), jnp.int32).at[:e].set(edge_index[1])
    srcp = srcp.reshape(NTILES, NCHUNK, CHUNK)
    dstp = dstp.reshape(NTILES, NCHUNK, CHUNK)
    w1 = W_res[0, :d]
    w2 = W_res[0, d:]
    b16 = jnp.full((LANES,), b_res[0], jnp.float32)

    v = _tc_vel(xp, W_vel.T, b_vel)
    for _ in range(ITERS):
        feat, p, q = _tc_prep(xp, W_lap.T, b_lap, w1, w2)
        adj, deg = _sc_edges(feat, p.reshape(NPAD), q.reshape(NPAD),
                             srcp, dstp, b16)
        xp, v = _tc_update(xp, v, feat, adj[0], adj[1],
                           deg[0].reshape(NPAD, 1), deg[1].reshape(NPAD, 1))
    return xp[:n]


# speedup vs baseline: 11.3175x; 1.0013x over previous
import functools

import jax
import jax.numpy as jnp
from jax import lax
from jax.experimental import pallas as pl
from jax.experimental.pallas import tpu as pltpu
from jax.experimental.pallas import tpu_sc as plsc

D = 128
EPS = 0.01
ITERS = 2

NCORES = 2
NSUB = 16
NTILES = NCORES * NSUB
CHUNK = 128
LANES = 16

N_NODES = 10000
N_EDGES = 320000
NPAD = 10240
ROWS_PER_SUB = NPAD // NSUB
SB = 16
NSB = 5
NCHUNK = SB * NSB
EPAD = NTILES * NCHUNK * CHUNK
BN = 1024


def _prep1_body(x_ref, wv_ref, bv_ref, wl_ref, bl_ref, w1_ref, w2_ref,
                v_ref, feat_ref, p_ref, q_ref):
    xb = x_ref[...]
    v_ref[...] = (
        jnp.dot(xb, wv_ref[...], preferred_element_type=jnp.float32)
        + bv_ref[...][None, :]
    )
    feat_ref[...] = (
        jnp.dot(xb, wl_ref[...], preferred_element_type=jnp.float32)
        + bl_ref[...][None, :]
    )
    p_ref[...] = jnp.sum(xb * w1_ref[...][None, :], axis=1, keepdims=True)
    q_ref[...] = jnp.sum(xb * w2_ref[...][None, :], axis=1, keepdims=True)


def _upprep_body(x_ref, v_ref, feat_ref, a0_ref, a1_ref, d0_ref, d1_ref,
                 wl_ref, bl_ref, w1_ref, w2_ref,
                 xo_ref, vo_ref, feato_ref, p_ref, q_ref):
    deg = d0_ref[...] + d1_ref[...]
    conv = deg * feat_ref[...] - a0_ref[...] - a1_ref[...]
    vn = v_ref[...] - EPS * conv
    vo_ref[...] = vn
    xn = x_ref[...] + EPS * vn
    xo_ref[...] = xn
    feato_ref[...] = (
        jnp.dot(xn, wl_ref[...], preferred_element_type=jnp.float32)
        + bl_ref[...][None, :]
    )
    p_ref[...] = jnp.sum(xn * w1_ref[...][None, :], axis=1, keepdims=True)
    q_ref[...] = jnp.sum(xn * w2_ref[...][None, :], axis=1, keepdims=True)


def _update_body(x_ref, v_ref, feat_ref, a0_ref, a1_ref, d0_ref, d1_ref,
                 xo_ref):
    deg = d0_ref[...] + d1_ref[...]
    conv = deg * feat_ref[...] - a0_ref[...] - a1_ref[...]
    vn = v_ref[...] - EPS * conv
    xo_ref[...] = x_ref[...] + EPS * vn


_ROW = pl.BlockSpec((BN, D), lambda i: (i, 0))
_COL = pl.BlockSpec((BN, 1), lambda i: (i, 0))
_WMAT = pl.BlockSpec((D, D), lambda i: (0, 0))
_WVEC = pl.BlockSpec((D,), lambda i: (0,))
_ROW_SHAPE = jax.ShapeDtypeStruct((NPAD, D), jnp.float32)
_COL_SHAPE = jax.ShapeDtypeStruct((NPAD, 1), jnp.float32)


def _tc_prep1(xp, wv_t, bv, wl_t, bl, w1, w2):
    return pl.pallas_call(
        _prep1_body,
        grid=(NPAD // BN,),
        in_specs=[_ROW, _WMAT, _WVEC, _WMAT, _WVEC, _WVEC, _WVEC],
        out_specs=[_ROW, _ROW, _COL, _COL],
        out_shape=[_ROW_SHAPE, _ROW_SHAPE, _COL_SHAPE, _COL_SHAPE],
    )(xp, wv_t, bv, wl_t, bl, w1, w2)


def _tc_upprep(xp, v, feat, adj0, adj1, deg0, deg1, wl_t, bl, w1, w2):
    return pl.pallas_call(
        _upprep_body,
        grid=(NPAD // BN,),
        in_specs=[_ROW, _ROW, _ROW, _ROW, _ROW, _COL, _COL,
                  _WMAT, _WVEC, _WVEC, _WVEC],
        out_specs=[_ROW, _ROW, _ROW, _COL, _COL],
        out_shape=[_ROW_SHAPE, _ROW_SHAPE, _ROW_SHAPE,
                   _COL_SHAPE, _COL_SHAPE],
    )(xp, v, feat, adj0, adj1, deg0, deg1, wl_t, bl, w1, w2)


def _tc_update(xp, v, feat, adj0, adj1, deg0, deg1):
    return pl.pallas_call(
        _update_body,
        grid=(NPAD // BN,),
        in_specs=[_ROW, _ROW, _ROW, _ROW, _ROW, _COL, _COL],
        out_specs=_ROW,
        out_shape=_ROW_SHAPE,
    )(xp, v, feat, adj0, adj1, deg0, deg1)


_MESH = plsc.VectorSubcoreMesh(
    core_axis_name="c", subcore_axis_name="s",
    num_cores=NCORES, num_subcores=NSUB,
)


@functools.partial(
    pl.kernel,
    out_type=(
        jax.ShapeDtypeStruct((NCORES, NPAD, D), jnp.float32),
        jax.ShapeDtypeStruct((NCORES, NPAD), jnp.float32),
    ),
    mesh=_MESH,
    compiler_params=pltpu.CompilerParams(needs_layout_passes=False),
    scratch_types=[
        pltpu.VMEM((SB, CHUNK), jnp.int32),
        pltpu.VMEM((SB, CHUNK), jnp.int32),
        pltpu.VMEM((SB, CHUNK), jnp.float32),
        pltpu.VMEM((SB, CHUNK), jnp.float32),
        pltpu.VMEM((SB, CHUNK), jnp.float32),
        pltpu.VMEM((LANES,), jnp.float32),
        pltpu.VMEM((CHUNK, D), jnp.float32),
        pltpu.VMEM((CHUNK, D), jnp.float32),
        pltpu.VMEM_SHARED((NPAD, D), jnp.float32),
        pltpu.VMEM_SHARED((NPAD,), jnp.float32),
        pltpu.SemaphoreType.DMA,
        pltpu.SemaphoreType.DMA,
        pltpu.SemaphoreType.DMA,
        pltpu.SemaphoreType.DMA,
    ],
)
def _sc_edges(feat_hbm, p_hbm, q_hbm, src_hbm, dst_hbm, b_hbm,
              adj_out, deg_out,
              src_v, dst_v, r_v, ps_v, qd_v, b_v, rows0, rows1,
              acc, dega, sem_g, sem_pq, sem_s, sem_d):
    c = lax.axis_index("c")
    s = lax.axis_index("s")
    wid = c * NSUB + s
    base = s * ROWS_PER_SUB

    zero16 = jnp.zeros((LANES,), jnp.float32)

    def zrow(e, carry):
        for d8 in range(D // LANES):
            rows0[e, pl.ds(d8 * LANES, LANES)] = zero16
        return carry

    lax.fori_loop(0, CHUNK, zrow, 0)

    for k in range(ROWS_PER_SUB // CHUNK):
        pltpu.sync_copy(rows0, acc.at[pl.ds(base + k * CHUNK, CHUNK)])
        pltpu.sync_copy(rows0.at[0],
                        dega.at[pl.ds(base + k * CHUNK, CHUNK)])

    pltpu.sync_copy(b_hbm, b_v)
    plsc.subcore_barrier()

    bb = b_v[...]
    bufs = (rows0, rows1)

    def start_gather(j, buf):
        pltpu.async_copy(feat_hbm.at[src_v.at[j]], buf, sem_g)

    def wait_gather(buf):
        pltpu.make_async_copy(feat_hbm.at[pl.ds(0, CHUNK)], buf, sem_g).wait()

    def wait_scatter(buf):
        pltpu.make_async_copy(feat_hbm.at[pl.ds(0, CHUNK)], buf, sem_s).wait()

    def sblock(b, carry):
        pltpu.sync_copy(src_hbm.at[wid, pl.ds(b * SB, SB)], src_v)
        pltpu.sync_copy(dst_hbm.at[wid, pl.ds(b * SB, SB)], dst_v)

        def active(j):
            return ((b * SB + j) * NTILES + wid) * CHUNK < N_EDGES

        pq = [pltpu.async_copy(p_hbm.at[src_v.at[js]], ps_v.at[js], sem_pq)
              for js in range(SB)]
        pq += [pltpu.async_copy(q_hbm.at[dst_v.at[js]], qd_v.at[js], sem_pq)
               for js in range(SB)]

        @pl.when(active(0))
        def _g0():
            start_gather(0, rows0)

        for dsc in pq:
            dsc.wait()

        def rstep(j, carry1):
            for k8 in range(CHUNK // LANES):
                sl = pl.ds(k8 * LANES, LANES)
                s16 = src_v[j, sl]
                d16 = dst_v[j, sl]
                rr = jnp.maximum(ps_v[j, sl] + qd_v[j, sl] + bb, 0.0)
                rr = jnp.where(s16 != d16, rr, 0.0)
                r_v[j, sl] = rr
            return carry1

        lax.fori_loop(0, SB, rstep, 0)

        for js in range(SB):
            pltpu.async_copy(r_v.at[js], dega.at[src_v.at[js]], sem_d,
                             add=True)

        def mpair(jj, carry1):
            for par in range(2):
                j = 2 * jj + par
                buf = bufs[par]
                other = bufs[1 - par]

                @pl.when(active(j))
                def _do_chunk():
                    wait_gather(buf)

                    @pl.when(b * SB + j >= 1)
                    def _drain_prev():
                        wait_scatter(other)

                    nxt = j + 1
                    if par == 0:
                        @pl.when(active(nxt))
                        def _gn():
                            start_gather(nxt, other)
                    else:
                        @pl.when((nxt < SB) & active(nxt))
                        def _gn():
                            start_gather(nxt, other)

                    @plsc.parallel_loop(0, CHUNK, unroll=4)
                    def _escale(e):
                        je = jnp.full((LANES,), j, jnp.int32)
                        ee = jnp.full((LANES,), e, jnp.int32)
                        rv = plsc.load_gather(r_v, [je, ee])
                        for d8 in range(D // LANES):
                            sl = pl.ds(d8 * LANES, LANES)
                            buf[e, sl] = buf[e, sl] * rv

                    pltpu.async_copy(buf, acc.at[dst_v.at[j]], sem_s,
                                     add=True)

            return carry1

        lax.fori_loop(0, SB // 2, mpair, 0)

        pltpu.make_async_copy(feat_hbm.at[pl.ds(0, SB)], r_v, sem_d).wait()
        return carry

    lax.fori_loop(0, NSB, sblock, 0)

    wait_scatter(rows0)

    plsc.subcore_barrier()

    pltpu.sync_copy(acc.at[pl.ds(base, ROWS_PER_SUB)],
                    adj_out.at[c].at[pl.ds(base, ROWS_PER_SUB)])
    pltpu.sync_copy(dega.at[pl.ds(base, ROWS_PER_SUB)],
                    deg_out.at[c].at[pl.ds(base, ROWS_PER_SUB)])


def kernel(x, edge_index, W_vel, b_vel, W_res, b_res, W_lap, b_lap):
    n, d = x.shape
    e = edge_index.shape[1]

    xp = jnp.zeros((NPAD, D), jnp.float32).at[:n].set(x)
    srcp = jnp.zeros((EPAD,), jnp.int32).at[:e].set(edge_index[0])
    dstp = jnp.zeros((EPAD,), jnp.int32).at[:e].set(edge_index[1])
    srcp = srcp.reshape(NCHUNK, NTILES, CHUNK).transpose(1, 0, 2)
    dstp = dstp.reshape(NCHUNK, NTILES, CHUNK).transpose(1, 0, 2)
    w1 = W_res[0, :d]
    w2 = W_res[0, d:]
    b16 = jnp.full((LANES,), b_res[0], jnp.float32)

    wl_t = W_lap.T
    v, feat, p, q = _tc_prep1(xp, W_vel.T, b_vel, wl_t, b_lap, w1, w2)
    for it in range(ITERS):
        adj, deg = _sc_edges(feat, p.reshape(NPAD), q.reshape(NPAD),
                             srcp, dstp, b16)
        d0 = deg[0].reshape(NPAD, 1)
        d1 = deg[1].reshape(NPAD, 1)
        if it < ITERS - 1:
            xp, v, feat, p, q = _tc_upprep(xp, v, feat, adj[0], adj[1],
                                           d0, d1, wl_t, b_lap, w1, w2)
        else:
            xp = _tc_update(xp, v, feat, adj[0], adj[1], d0, d1)
    return xp[:n]


# speedup vs baseline: 13.8759x; 1.2261x over previous
import functools

import jax
import jax.numpy as jnp
from jax import lax
from jax.experimental import pallas as pl
from jax.experimental.pallas import tpu as pltpu
from jax.experimental.pallas import tpu_sc as plsc

D = 128
EPS = 0.01
ITERS = 2

NCORES = 2
NSUB = 16
NTILES = NCORES * NSUB
CHUNK = 128
LANES = 16

N_NODES = 10000
N_EDGES = 320000
NPAD = 10240
ROWS_PER_SUB = NPAD // NSUB
SB = 16
NSB = 5
NCHUNK = SB * NSB
EPAD = NTILES * NCHUNK * CHUNK
BN = 1024


def _prep1_body(x_ref, wv_ref, bv_ref, wl_ref, bl_ref, w1_ref, w2_ref,
                v_ref, feat_ref, p_ref, q_ref):
    xb = x_ref[...]
    v_ref[...] = (
        jnp.dot(xb, wv_ref[...], preferred_element_type=jnp.float32)
        + bv_ref[...][None, :]
    )
    feat_ref[...] = (
        jnp.dot(xb, wl_ref[...], preferred_element_type=jnp.float32)
        + bl_ref[...][None, :]
    )
    p_ref[...] = jnp.sum(xb * w1_ref[...][None, :], axis=1, keepdims=True)
    q_ref[...] = jnp.sum(xb * w2_ref[...][None, :], axis=1, keepdims=True)


def _upprep_body(x_ref, v_ref, feat_ref, a0_ref, a1_ref, d0_ref, d1_ref,
                 wl_ref, bl_ref, w1_ref, w2_ref,
                 xo_ref, vo_ref, feato_ref, p_ref, q_ref):
    deg = d0_ref[...] + d1_ref[...]
    conv = deg * feat_ref[...] - a0_ref[...] - a1_ref[...]
    vn = v_ref[...] - EPS * conv
    vo_ref[...] = vn
    xn = x_ref[...] + EPS * vn
    xo_ref[...] = xn
    feato_ref[...] = (
        jnp.dot(xn, wl_ref[...], preferred_element_type=jnp.float32)
        + bl_ref[...][None, :]
    )
    p_ref[...] = jnp.sum(xn * w1_ref[...][None, :], axis=1, keepdims=True)
    q_ref[...] = jnp.sum(xn * w2_ref[...][None, :], axis=1, keepdims=True)


def _update_body(x_ref, v_ref, feat_ref, a0_ref, a1_ref, d0_ref, d1_ref,
                 xo_ref):
    deg = d0_ref[...] + d1_ref[...]
    conv = deg * feat_ref[...] - a0_ref[...] - a1_ref[...]
    vn = v_ref[...] - EPS * conv
    xo_ref[...] = x_ref[...] + EPS * vn


_ROW = pl.BlockSpec((BN, D), lambda i: (i, 0))
_COL = pl.BlockSpec((BN, 1), lambda i: (i, 0))
_WMAT = pl.BlockSpec((D, D), lambda i: (0, 0))
_WVEC = pl.BlockSpec((D,), lambda i: (0,))
_ROW_SHAPE = jax.ShapeDtypeStruct((NPAD, D), jnp.float32)
_COL_SHAPE = jax.ShapeDtypeStruct((NPAD, 1), jnp.float32)


def _tc_prep1(xp, wv_t, bv, wl_t, bl, w1, w2):
    return pl.pallas_call(
        _prep1_body,
        grid=(NPAD // BN,),
        in_specs=[_ROW, _WMAT, _WVEC, _WMAT, _WVEC, _WVEC, _WVEC],
        out_specs=[_ROW, _ROW, _COL, _COL],
        out_shape=[_ROW_SHAPE, _ROW_SHAPE, _COL_SHAPE, _COL_SHAPE],
    )(xp, wv_t, bv, wl_t, bl, w1, w2)


def _tc_upprep(xp, v, feat, adj0, adj1, deg0, deg1, wl_t, bl, w1, w2):
    return pl.pallas_call(
        _upprep_body,
        grid=(NPAD // BN,),
        in_specs=[_ROW, _ROW, _ROW, _ROW, _ROW, _COL, _COL,
                  _WMAT, _WVEC, _WVEC, _WVEC],
        out_specs=[_ROW, _ROW, _ROW, _COL, _COL],
        out_shape=[_ROW_SHAPE, _ROW_SHAPE, _ROW_SHAPE,
                   _COL_SHAPE, _COL_SHAPE],
    )(xp, v, feat, adj0, adj1, deg0, deg1, wl_t, bl, w1, w2)


def _tc_update(xp, v, feat, adj0, adj1, deg0, deg1):
    return pl.pallas_call(
        _update_body,
        grid=(NPAD // BN,),
        in_specs=[_ROW, _ROW, _ROW, _ROW, _ROW, _COL, _COL],
        out_specs=_ROW,
        out_shape=_ROW_SHAPE,
    )(xp, v, feat, adj0, adj1, deg0, deg1)


_MESH = plsc.VectorSubcoreMesh(
    core_axis_name="c", subcore_axis_name="s",
    num_cores=NCORES, num_subcores=NSUB,
)


@functools.partial(
    pl.kernel,
    out_type=(
        jax.ShapeDtypeStruct((NCORES, NPAD, D), jnp.float32),
        jax.ShapeDtypeStruct((NCORES, NPAD), jnp.float32),
    ),
    mesh=_MESH,
    compiler_params=pltpu.CompilerParams(needs_layout_passes=False),
    scratch_types=[
        pltpu.VMEM((SB, CHUNK), jnp.int32),
        pltpu.VMEM((SB, CHUNK), jnp.int32),
        pltpu.VMEM((SB, CHUNK), jnp.float32),
        pltpu.VMEM((SB, CHUNK), jnp.float32),
        pltpu.VMEM((LANES,), jnp.float32),
        pltpu.VMEM((SB * CHUNK,), jnp.int32),
        pltpu.VMEM((SB * CHUNK,), jnp.int32),
        pltpu.VMEM((SB * CHUNK,), jnp.float32),
        pltpu.VMEM((1, CHUNK), jnp.int32),
        pltpu.VMEM((1, CHUNK), jnp.int32),
        pltpu.VMEM((CHUNK, D), jnp.float32),
        pltpu.VMEM((CHUNK, D), jnp.float32),
        pltpu.VMEM_SHARED((NPAD, D), jnp.float32),
        pltpu.VMEM_SHARED((NPAD,), jnp.float32),
        pltpu.SemaphoreType.DMA,
        pltpu.SemaphoreType.DMA,
        pltpu.SemaphoreType.DMA,
        pltpu.SemaphoreType.DMA,
    ],
)
def _sc_edges(feat_hbm, p_hbm, q_hbm, src_hbm, dst_hbm, b_hbm,
              adj_out, deg_out,
              src_v, dst_v, r_v, qd_v, b_v, scmp, dcmp, rcmp,
              dstrow0, dstrow1, rows0, rows1,
              acc, dega, sem_g, sem_pq, sem_s, sem_d):
    c = lax.axis_index("c")
    s = lax.axis_index("s")
    wid = c * NSUB + s
    base = s * ROWS_PER_SUB

    zero16 = jnp.zeros((LANES,), jnp.float32)

    def zrow(e, carry):
        for d8 in range(D // LANES):
            rows0[e, pl.ds(d8 * LANES, LANES)] = zero16
        return carry

    lax.fori_loop(0, CHUNK, zrow, 0)

    for k in range(ROWS_PER_SUB // CHUNK):
        pltpu.sync_copy(rows0, acc.at[pl.ds(base + k * CHUNK, CHUNK)])
        pltpu.sync_copy(rows0.at[0],
                        dega.at[pl.ds(base + k * CHUNK, CHUNK)])

    pltpu.sync_copy(b_hbm, b_v)
    plsc.subcore_barrier()

    bb = b_v[...]
    bufs = (rows0, rows1)
    dstrows = (dstrow0, dstrow1)

    def start_gather(j, buf):
        jb = pl.multiple_of(j * CHUNK, CHUNK)
        pltpu.async_copy(feat_hbm.at[scmp.at[pl.ds(jb, CHUNK)]],
                         buf, sem_g)

    def wait_gather(buf):
        pltpu.make_async_copy(feat_hbm.at[pl.ds(0, CHUNK)], buf, sem_g).wait()

    def wait_scatter(buf):
        pltpu.make_async_copy(feat_hbm.at[pl.ds(0, CHUNK)], buf, sem_s).wait()

    def sblock(b, pend):
        pltpu.sync_copy(src_hbm.at[wid, b], src_v)
        pltpu.sync_copy(dst_hbm.at[wid, b], dst_v)

        pq = [pltpu.async_copy(p_hbm.at[src_v.at[js]], r_v.at[js], sem_pq)
              for js in range(SB)]
        pq += [pltpu.async_copy(q_hbm.at[dst_v.at[js]], qd_v.at[js], sem_pq)
               for js in range(SB)]
        for dsc in pq:
            dsc.wait()

        def rstep(j, carry1):
            for k8 in range(CHUNK // LANES):
                sl = pl.ds(k8 * LANES, LANES)
                s16 = src_v[j, sl]
                d16 = dst_v[j, sl]
                rr = jnp.maximum(r_v[j, sl] + qd_v[j, sl] + bb, 0.0)
                rr = jnp.where(s16 != d16, rr, 0.0)
                r_v[j, sl] = rr
            return carry1

        lax.fori_loop(0, SB, rstep, 0)

        for js in range(SB):
            pltpu.async_copy(r_v.at[js], dega.at[src_v.at[js]], sem_d,
                             add=True)

        zlanes = jnp.zeros((LANES,), jnp.float32)
        iota16 = lax.iota(jnp.int32, LANES)
        for i in range(SB * CHUNK // LANES):
            idv = N_NODES + (i % (CHUNK // LANES)) * LANES + iota16
            sl = pl.ds(i * LANES, LANES)
            scmp[sl] = idv
            dcmp[sl] = idv
            rcmp[sl] = zlanes

        def cstep(j, off1):
            def ck(k8, off2):
                off2 = pl.multiple_of(off2, 8)
                sl = pl.ds(k8 * LANES, LANES)
                s16 = src_v[j, sl]
                d16 = dst_v[j, sl]
                rr = r_v[j, sl]
                m = rr > 0.0
                plsc.store_compressed(scmp.at[pl.ds(off2, LANES)], s16,
                                      mask=m)
                plsc.store_compressed(dcmp.at[pl.ds(off2, LANES)], d16,
                                      mask=m)
                plsc.store_compressed(rcmp.at[pl.ds(off2, LANES)], rr,
                                      mask=m)
                cnt = jnp.sum(m.astype(jnp.int32))
                return off2 + ((cnt + 7) // 8) * 8
            return lax.fori_loop(0, CHUNK // LANES, ck, off1)

        off = lax.fori_loop(0, SB, cstep, jnp.int32(0))
        nact = (off + CHUNK - 1) // CHUNK

        @pl.when((nact > 0) & (pend == 1))
        def _drain_xblock():
            wait_scatter(rows0)

        @pl.when(nact > 0)
        def _g0():
            start_gather(0, rows0)

        def mpair(jj, carry1):
            for par in range(2):
                j = 2 * jj + par
                buf = bufs[par]
                other = bufs[1 - par]
                dstrow = dstrows[par]

                @pl.when(j < nact)
                def _do_chunk():
                    wait_gather(buf)

                    @pl.when(j >= 1)
                    def _drain_prev():
                        wait_scatter(other)

                    @pl.when(j + 1 < nact)
                    def _gn():
                        start_gather(j + 1, other)

                    for k8 in range(CHUNK // LANES):
                        sl = pl.ds(k8 * LANES, LANES)
                        dsrc = pl.multiple_of(j * CHUNK + k8 * LANES, 8)
                        dstrow[0, sl] = dcmp[pl.ds(dsrc, LANES)]

                    @plsc.parallel_loop(0, CHUNK, unroll=4)
                    def _escale(e):
                        ee = jnp.full((LANES,), j * CHUNK + e, jnp.int32)
                        rv = plsc.load_gather(rcmp, [ee])
                        for d8 in range(D // LANES):
                            sl = pl.ds(d8 * LANES, LANES)
                            buf[e, sl] = buf[e, sl] * rv

                    pltpu.async_copy(buf, acc.at[dstrow.at[0]], sem_s,
                                     add=True)

            return carry1

        lax.fori_loop(0, SB // 2, mpair, 0)

        pltpu.make_async_copy(feat_hbm.at[pl.ds(0, SB)], r_v, sem_d).wait()
        return jnp.where(nact > 0, jnp.int32(1), pend)

    pend_f = lax.fori_loop(0, NSB, sblock, jnp.int32(0))

    @pl.when(pend_f == 1)
    def _drain_last():
        wait_scatter(rows0)

    plsc.subcore_barrier()

    pltpu.sync_copy(acc.at[pl.ds(base, ROWS_PER_SUB)],
                    adj_out.at[c].at[pl.ds(base, ROWS_PER_SUB)])
    pltpu.sync_copy(dega.at[pl.ds(base, ROWS_PER_SUB)],
                    deg_out.at[c].at[pl.ds(base, ROWS_PER_SUB)])


def kernel(x, edge_index, W_vel, b_vel, W_res, b_res, W_lap, b_lap):
    n, d = x.shape
    e = edge_index.shape[1]

    xp = jnp.zeros((NPAD, D), jnp.float32).at[:n].set(x)
    pad_ids = jnp.arange(EPAD - e, dtype=jnp.int32) % n
    srcp = jnp.concatenate([edge_index[0], pad_ids])
    dstp = jnp.concatenate([edge_index[1], pad_ids])
    srcp = srcp.reshape(NCHUNK, NTILES, CHUNK).transpose(1, 0, 2)
    dstp = dstp.reshape(NCHUNK, NTILES, CHUNK).transpose(1, 0, 2)
    srcp = srcp.reshape(NTILES, NSB, SB, CHUNK)
    dstp = dstp.reshape(NTILES, NSB, SB, CHUNK)
    w1 = W_res[0, :d]
    w2 = W_res[0, d:]
    b16 = jnp.full((LANES,), b_res[0], jnp.float32)

    wl_t = W_lap.T
    v, feat, p, q = _tc_prep1(xp, W_vel.T, b_vel, wl_t, b_lap, w1, w2)
    for it in range(ITERS):
        adj, deg = _sc_edges(feat, p.reshape(NPAD), q.reshape(NPAD),
                             srcp, dstp, b16)
        d0 = deg[0].reshape(NPAD, 1)
        d1 = deg[1].reshape(NPAD, 1)
        if it < ITERS - 1:
            xp, v, feat, p, q = _tc_upprep(xp, v, feat, adj[0], adj[1],
                                           d0, d1, wl_t, b_lap, w1, w2)
        else:
            xp = _tc_update(xp, v, feat, adj[0], adj[1], d0, d1)
    return xp[:n]


# speedup vs baseline: 15.9922x; 1.1525x over previous
import functools

import jax
import jax.numpy as jnp
from jax import lax
from jax.experimental import pallas as pl
from jax.experimental.pallas import tpu as pltpu
from jax.experimental.pallas import tpu_sc as plsc

D = 128
EPS = 0.01
ITERS = 2

NCORES = 2
NSUB = 16
NTILES = NCORES * NSUB
CHUNK = 128
LANES = 16

N_NODES = 10000
N_EDGES = 320000
NPAD = 10240
ROWS_PER_SUB = NPAD // NSUB
SB = 16
NSB = 5
NCHUNK = SB * NSB
EPAD = NTILES * NCHUNK * CHUNK
BN = 1024


def _prep1_body(x_ref, wv_ref, bv_ref, wl_ref, bl_ref, w1_ref, w2_ref,
                v_ref, feat_ref, p_ref, q_ref):
    xb = x_ref[...]
    v_ref[...] = (
        jnp.dot(xb, wv_ref[...], preferred_element_type=jnp.float32)
        + bv_ref[...][None, :]
    )
    feat_ref[...] = (
        jnp.dot(xb, wl_ref[...], preferred_element_type=jnp.float32)
        + bl_ref[...][None, :]
    )
    p_ref[...] = jnp.sum(xb * w1_ref[...][None, :], axis=1, keepdims=True)
    q_ref[...] = jnp.sum(xb * w2_ref[...][None, :], axis=1, keepdims=True)


def _upprep_body(x_ref, v_ref, feat_ref, a0_ref, a1_ref, d0_ref, d1_ref,
                 wl_ref, bl_ref, w1_ref, w2_ref,
                 xo_ref, vo_ref, feato_ref, p_ref, q_ref):
    deg = d0_ref[...] + d1_ref[...]
    conv = deg * feat_ref[...] - a0_ref[...] - a1_ref[...]
    vn = v_ref[...] - EPS * conv
    vo_ref[...] = vn
    xn = x_ref[...] + EPS * vn
    xo_ref[...] = xn
    feato_ref[...] = (
        jnp.dot(xn, wl_ref[...], preferred_element_type=jnp.float32)
        + bl_ref[...][None, :]
    )
    p_ref[...] = jnp.sum(xn * w1_ref[...][None, :], axis=1, keepdims=True)
    q_ref[...] = jnp.sum(xn * w2_ref[...][None, :], axis=1, keepdims=True)


def _update_body(x_ref, v_ref, feat_ref, a0_ref, a1_ref, d0_ref, d1_ref,
                 xo_ref):
    deg = d0_ref[...] + d1_ref[...]
    conv = deg * feat_ref[...] - a0_ref[...] - a1_ref[...]
    vn = v_ref[...] - EPS * conv
    xo_ref[...] = x_ref[...] + EPS * vn


_ROW = pl.BlockSpec((BN, D), lambda i: (i, 0))
_COL = pl.BlockSpec((BN, 1), lambda i: (i, 0))
_WMAT = pl.BlockSpec((D, D), lambda i: (0, 0))
_WVEC = pl.BlockSpec((D,), lambda i: (0,))
_ROW_SHAPE = jax.ShapeDtypeStruct((NPAD, D), jnp.float32)
_COL_SHAPE = jax.ShapeDtypeStruct((NPAD, 1), jnp.float32)


def _tc_prep1(xp, wv_t, bv, wl_t, bl, w1, w2):
    return pl.pallas_call(
        _prep1_body,
        grid=(NPAD // BN,),
        in_specs=[_ROW, _WMAT, _WVEC, _WMAT, _WVEC, _WVEC, _WVEC],
        out_specs=[_ROW, _ROW, _COL, _COL],
        out_shape=[_ROW_SHAPE, _ROW_SHAPE, _COL_SHAPE, _COL_SHAPE],
    )(xp, wv_t, bv, wl_t, bl, w1, w2)


def _tc_upprep(xp, v, feat, adj0, adj1, deg0, deg1, wl_t, bl, w1, w2):
    return pl.pallas_call(
        _upprep_body,
        grid=(NPAD // BN,),
        in_specs=[_ROW, _ROW, _ROW, _ROW, _ROW, _COL, _COL,
                  _WMAT, _WVEC, _WVEC, _WVEC],
        out_specs=[_ROW, _ROW, _ROW, _COL, _COL],
        out_shape=[_ROW_SHAPE, _ROW_SHAPE, _ROW_SHAPE,
                   _COL_SHAPE, _COL_SHAPE],
    )(xp, v, feat, adj0, adj1, deg0, deg1, wl_t, bl, w1, w2)


def _tc_update(xp, v, feat, adj0, adj1, deg0, deg1):
    return pl.pallas_call(
        _update_body,
        grid=(NPAD // BN,),
        in_specs=[_ROW, _ROW, _ROW, _ROW, _ROW, _COL, _COL],
        out_specs=_ROW,
        out_shape=_ROW_SHAPE,
    )(xp, v, feat, adj0, adj1, deg0, deg1)


_MESH = plsc.VectorSubcoreMesh(
    core_axis_name="c", subcore_axis_name="s",
    num_cores=NCORES, num_subcores=NSUB,
)


@functools.partial(
    pl.kernel,
    out_type=(
        jax.ShapeDtypeStruct((NCORES, NPAD, D), jnp.float32),
        jax.ShapeDtypeStruct((NCORES, NPAD), jnp.float32),
    ),
    mesh=_MESH,
    compiler_params=pltpu.CompilerParams(needs_layout_passes=False),
    scratch_types=[
        pltpu.VMEM((SB, CHUNK), jnp.int32),
        pltpu.VMEM((SB, CHUNK), jnp.int32),
        pltpu.VMEM((SB, CHUNK), jnp.float32),
        pltpu.VMEM((SB, CHUNK), jnp.float32),
        pltpu.VMEM((LANES,), jnp.float32),
        pltpu.VMEM((SB * CHUNK,), jnp.int32),
        pltpu.VMEM((SB * CHUNK,), jnp.int32),
        pltpu.VMEM((SB * CHUNK,), jnp.float32),
        pltpu.VMEM((1, CHUNK), jnp.int32),
        pltpu.VMEM((1, CHUNK), jnp.int32),
        pltpu.VMEM((CHUNK, D), jnp.float32),
        pltpu.VMEM((CHUNK, D), jnp.float32),
        pltpu.VMEM_SHARED((NPAD, D), jnp.float32),
        pltpu.VMEM_SHARED((NPAD,), jnp.float32),
        pltpu.SemaphoreType.DMA,
        pltpu.SemaphoreType.DMA,
        pltpu.SemaphoreType.DMA,
        pltpu.SemaphoreType.DMA,
    ],
)
def _sc_edges(feat_hbm, p_hbm, q_hbm, src_hbm, dst_hbm, b_hbm,
              adj_out, deg_out,
              src_v, dst_v, r_v, qd_v, b_v, scmp, dcmp, rcmp,
              dstrow0, dstrow1, rows0, rows1,
              acc, dega, sem_g, sem_pq, sem_s, sem_d):
    c = lax.axis_index("c")
    s = lax.axis_index("s")
    wid = c * NSUB + s
    base = s * ROWS_PER_SUB

    zero16 = jnp.zeros((LANES,), jnp.float32)

    def zrow(e, carry):
        for d8 in range(D // LANES):
            rows0[e, pl.ds(d8 * LANES, LANES)] = zero16
        return carry

    lax.fori_loop(0, CHUNK, zrow, 0)

    for k in range(ROWS_PER_SUB // CHUNK):
        pltpu.sync_copy(rows0, acc.at[pl.ds(base + k * CHUNK, CHUNK)])
        pltpu.sync_copy(rows0.at[0],
                        dega.at[pl.ds(base + k * CHUNK, CHUNK)])

    pltpu.sync_copy(b_hbm, b_v)
    plsc.subcore_barrier()

    bb = b_v[...]
    bufs = (rows0, rows1)
    dstrows = (dstrow0, dstrow1)

    def start_gather(j, buf):
        jb = pl.multiple_of(j * CHUNK, CHUNK)
        pltpu.async_copy(feat_hbm.at[scmp.at[pl.ds(jb, CHUNK)]],
                         buf, sem_g)

    def wait_gather(buf):
        pltpu.make_async_copy(feat_hbm.at[pl.ds(0, CHUNK)], buf, sem_g).wait()

    def wait_scatter(buf):
        pltpu.make_async_copy(feat_hbm.at[pl.ds(0, CHUNK)], buf, sem_s).wait()

    def sblock(b, pend):
        pltpu.sync_copy(src_hbm.at[wid, b], src_v)
        pltpu.sync_copy(dst_hbm.at[wid, b], dst_v)

        pq = [pltpu.async_copy(p_hbm.at[src_v.at[js]], r_v.at[js], sem_pq)
              for js in range(SB)]
        pq += [pltpu.async_copy(q_hbm.at[dst_v.at[js]], qd_v.at[js], sem_pq)
               for js in range(SB)]
        for dsc in pq:
            dsc.wait()

        def rstep(j, carry1):
            for k8 in range(CHUNK // LANES):
                sl = pl.ds(k8 * LANES, LANES)
                s16 = src_v[j, sl]
                d16 = dst_v[j, sl]
                rr = jnp.maximum(r_v[j, sl] + qd_v[j, sl] + bb, 0.0)
                rr = jnp.where(s16 != d16, rr, 0.0)
                r_v[j, sl] = rr
            return carry1

        lax.fori_loop(0, SB, rstep, 0)

        for js in range(SB):
            pltpu.async_copy(r_v.at[js], dega.at[src_v.at[js]], sem_d,
                             add=True)

        zlanes = jnp.zeros((LANES,), jnp.float32)
        iota16 = lax.iota(jnp.int32, LANES)
        for i in range(SB * CHUNK // LANES):
            idv = N_NODES + (i % (CHUNK // LANES)) * LANES + iota16
            sl = pl.ds(i * LANES, LANES)
            scmp[sl] = idv
            dcmp[sl] = idv
            rcmp[sl] = zlanes

        def cstep(j, off1):
            def ck(k8, off2):
                sl = pl.ds(k8 * LANES, LANES)
                s16 = src_v[j, sl]
                d16 = dst_v[j, sl]
                rr = r_v[j, sl]
                m = rr > 0.0
                plsc.store_compressed(scmp.at[pl.ds(off2, LANES)], s16,
                                      mask=m)
                plsc.store_compressed(dcmp.at[pl.ds(off2, LANES)], d16,
                                      mask=m)
                plsc.store_compressed(rcmp.at[pl.ds(off2, LANES)], rr,
                                      mask=m)
                return off2 + jnp.sum(m.astype(jnp.int32))
            return lax.fori_loop(0, CHUNK // LANES, ck, off1)

        off = lax.fori_loop(0, SB, cstep, jnp.int32(0))
        nact = (off + CHUNK - 1) // CHUNK

        @pl.when((nact > 0) & (pend == 1))
        def _drain_xblock():
            wait_scatter(rows0)

        @pl.when(nact > 0)
        def _g0():
            start_gather(0, rows0)

        def mpair(jj, carry1):
            for par in range(2):
                j = 2 * jj + par
                buf = bufs[par]
                other = bufs[1 - par]
                dstrow = dstrows[par]

                @pl.when(j < nact)
                def _do_chunk():
                    wait_gather(buf)

                    @pl.when(j >= 1)
                    def _drain_prev():
                        wait_scatter(other)

                    @pl.when(j + 1 < nact)
                    def _gn():
                        start_gather(j + 1, other)

                    for k8 in range(CHUNK // LANES):
                        sl = pl.ds(k8 * LANES, LANES)
                        dsrc = pl.multiple_of(j * CHUNK + k8 * LANES, 8)
                        dstrow[0, sl] = dcmp[pl.ds(dsrc, LANES)]

                    @plsc.parallel_loop(0, CHUNK, unroll=4)
                    def _escale(e):
                        ee = jnp.full((LANES,), j * CHUNK + e, jnp.int32)
                        rv = plsc.load_gather(rcmp, [ee])
                        for d8 in range(D // LANES):
                            sl = pl.ds(d8 * LANES, LANES)
                            buf[e, sl] = buf[e, sl] * rv

                    pltpu.async_copy(buf, acc.at[dstrow.at[0]], sem_s,
                                     add=True)

            return carry1

        lax.fori_loop(0, SB // 2, mpair, 0)

        pltpu.make_async_copy(feat_hbm.at[pl.ds(0, SB)], r_v, sem_d).wait()
        return jnp.where(nact > 0, jnp.int32(1), pend)

    pend_f = lax.fori_loop(0, NSB, sblock, jnp.int32(0))

    @pl.when(pend_f == 1)
    def _drain_last():
        wait_scatter(rows0)

    plsc.subcore_barrier()

    pltpu.sync_copy(acc.at[pl.ds(base, ROWS_PER_SUB)],
                    adj_out.at[c].at[pl.ds(base, ROWS_PER_SUB)])
    pltpu.sync_copy(dega.at[pl.ds(base, ROWS_PER_SUB)],
                    deg_out.at[c].at[pl.ds(base, ROWS_PER_SUB)])


def kernel(x, edge_index, W_vel, b_vel, W_res, b_res, W_lap, b_lap):
    n, d = x.shape
    e = edge_index.shape[1]

    xp = jnp.zeros((NPAD, D), jnp.float32).at[:n].set(x)
    pad_ids = jnp.arange(EPAD - e, dtype=jnp.int32) % n
    srcp = jnp.concatenate([edge_index[0], pad_ids])
    dstp = jnp.concatenate([edge_index[1], pad_ids])
    srcp = srcp.reshape(NCHUNK, NTILES, CHUNK).transpose(1, 0, 2)
    dstp = dstp.reshape(NCHUNK, NTILES, CHUNK).transpose(1, 0, 2)
    srcp = srcp.reshape(NTILES, NSB, SB, CHUNK)
    dstp = dstp.reshape(NTILES, NSB, SB, CHUNK)
    w1 = W_res[0, :d]
    w2 = W_res[0, d:]
    b16 = jnp.full((LANES,), b_res[0], jnp.float32)

    wl_t = W_lap.T
    v, feat, p, q = _tc_prep1(xp, W_vel.T, b_vel, wl_t, b_lap, w1, w2)
    for it in range(ITERS):
        adj, deg = _sc_edges(feat, p.reshape(NPAD), q.reshape(NPAD),
                             srcp, dstp, b16)
        d0 = deg[0].reshape(NPAD, 1)
        d1 = deg[1].reshape(NPAD, 1)
        if it < ITERS - 1:
            xp, v, feat, p, q = _tc_upprep(xp, v, feat, adj[0], adj[1],
                                           d0, d1, wl_t, b_lap, w1, w2)
        else:
            xp = _tc_update(xp, v, feat, adj[0], adj[1], d0, d1)
    return xp[:n]


# speedup vs baseline: 16.1831x; 1.0119x over previous
import functools

import jax
import jax.numpy as jnp
from jax import lax
from jax.experimental import pallas as pl
from jax.experimental.pallas import tpu as pltpu
from jax.experimental.pallas import tpu_sc as plsc

D = 128
EPS = 0.01
ITERS = 2

NCORES = 2
NSUB = 16
NTILES = NCORES * NSUB
CHUNK = 128
LANES = 16

N_NODES = 10000
N_EDGES = 320000
NPAD = 10240
ROWS_PER_SUB = NPAD // NSUB
SB = 16
NSB = 5
NCHUNK = SB * NSB
EPAD = NTILES * NCHUNK * CHUNK
BN = 1024


def _prep1_body(x_ref, wv_ref, bv_ref, wl_ref, bl_ref, w1_ref, w2_ref,
                v_ref, feat_ref, p_ref, q_ref):
    xb = x_ref[...]
    v_ref[...] = (
        jnp.dot(xb, wv_ref[...], preferred_element_type=jnp.float32)
        + bv_ref[...][None, :]
    )
    feat_ref[...] = (
        jnp.dot(xb, wl_ref[...], preferred_element_type=jnp.float32)
        + bl_ref[...][None, :]
    )
    p_ref[...] = jnp.sum(xb * w1_ref[...][None, :], axis=1, keepdims=True)
    q_ref[...] = jnp.sum(xb * w2_ref[...][None, :], axis=1, keepdims=True)


def _upprep_body(x_ref, v_ref, feat_ref, a0_ref, a1_ref, d0_ref, d1_ref,
                 wl_ref, bl_ref, w1_ref, w2_ref,
                 xo_ref, vo_ref, feato_ref, p_ref, q_ref):
    deg = d0_ref[...] + d1_ref[...]
    conv = deg * feat_ref[...] - a0_ref[...] - a1_ref[...]
    vn = v_ref[...] - EPS * conv
    vo_ref[...] = vn
    xn = x_ref[...] + EPS * vn
    xo_ref[...] = xn
    feato_ref[...] = (
        jnp.dot(xn, wl_ref[...], preferred_element_type=jnp.float32)
        + bl_ref[...][None, :]
    )
    p_ref[...] = jnp.sum(xn * w1_ref[...][None, :], axis=1, keepdims=True)
    q_ref[...] = jnp.sum(xn * w2_ref[...][None, :], axis=1, keepdims=True)


def _update_body(x_ref, v_ref, feat_ref, a0_ref, a1_ref, d0_ref, d1_ref,
                 xo_ref):
    deg = d0_ref[...] + d1_ref[...]
    conv = deg * feat_ref[...] - a0_ref[...] - a1_ref[...]
    vn = v_ref[...] - EPS * conv
    xo_ref[...] = x_ref[...] + EPS * vn


_ROW = pl.BlockSpec((BN, D), lambda i: (i, 0))
_COL = pl.BlockSpec((BN, 1), lambda i: (i, 0))
_WMAT = pl.BlockSpec((D, D), lambda i: (0, 0))
_WVEC = pl.BlockSpec((D,), lambda i: (0,))
_ROW_SHAPE = jax.ShapeDtypeStruct((NPAD, D), jnp.float32)
_COL_SHAPE = jax.ShapeDtypeStruct((NPAD, 1), jnp.float32)


def _tc_prep1(xp, wv_t, bv, wl_t, bl, w1, w2):
    return pl.pallas_call(
        _prep1_body,
        grid=(NPAD // BN,),
        in_specs=[_ROW, _WMAT, _WVEC, _WMAT, _WVEC, _WVEC, _WVEC],
        out_specs=[_ROW, _ROW, _COL, _COL],
        out_shape=[_ROW_SHAPE, _ROW_SHAPE, _COL_SHAPE, _COL_SHAPE],
    )(xp, wv_t, bv, wl_t, bl, w1, w2)


def _tc_upprep(xp, v, feat, adj0, adj1, deg0, deg1, wl_t, bl, w1, w2):
    return pl.pallas_call(
        _upprep_body,
        grid=(NPAD // BN,),
        in_specs=[_ROW, _ROW, _ROW, _ROW, _ROW, _COL, _COL,
                  _WMAT, _WVEC, _WVEC, _WVEC],
        out_specs=[_ROW, _ROW, _ROW, _COL, _COL],
        out_shape=[_ROW_SHAPE, _ROW_SHAPE, _ROW_SHAPE,
                   _COL_SHAPE, _COL_SHAPE],
    )(xp, v, feat, adj0, adj1, deg0, deg1, wl_t, bl, w1, w2)


def _tc_update(xp, v, feat, adj0, adj1, deg0, deg1):
    return pl.pallas_call(
        _update_body,
        grid=(NPAD // BN,),
        in_specs=[_ROW, _ROW, _ROW, _ROW, _ROW, _COL, _COL],
        out_specs=_ROW,
        out_shape=_ROW_SHAPE,
    )(xp, v, feat, adj0, adj1, deg0, deg1)


_MESH = plsc.VectorSubcoreMesh(
    core_axis_name="c", subcore_axis_name="s",
    num_cores=NCORES, num_subcores=NSUB,
)


@functools.partial(
    pl.kernel,
    out_type=(
        jax.ShapeDtypeStruct((NCORES, NPAD, D), jnp.float32),
        jax.ShapeDtypeStruct((NCORES, NPAD), jnp.float32),
    ),
    mesh=_MESH,
    compiler_params=pltpu.CompilerParams(needs_layout_passes=False),
    scratch_types=[
        pltpu.VMEM((SB, CHUNK), jnp.int32),
        pltpu.VMEM((SB, CHUNK), jnp.int32),
        pltpu.VMEM((SB, CHUNK), jnp.float32),
        pltpu.VMEM((SB, CHUNK), jnp.float32),
        pltpu.VMEM((LANES,), jnp.float32),
        pltpu.VMEM((SB * CHUNK,), jnp.int32),
        pltpu.VMEM((SB * CHUNK,), jnp.int32),
        pltpu.VMEM((SB * CHUNK,), jnp.float32),
        pltpu.VMEM((1, CHUNK), jnp.int32),
        pltpu.VMEM((1, CHUNK), jnp.int32),
        pltpu.VMEM((CHUNK, D), jnp.float32),
        pltpu.VMEM((CHUNK, D), jnp.float32),
        pltpu.VMEM_SHARED((NPAD, D), jnp.float32),
        pltpu.VMEM_SHARED((NPAD,), jnp.float32),
        pltpu.SemaphoreType.DMA,
        pltpu.SemaphoreType.DMA,
        pltpu.SemaphoreType.DMA,
        pltpu.SemaphoreType.DMA,
    ],
)
def _sc_edges(feat_hbm, p_hbm, q_hbm, src_hbm, dst_hbm, b_hbm,
              adj_out, deg_out,
              src_v, dst_v, r_v, qd_v, b_v, scmp, dcmp, rcmp,
              dstrow0, dstrow1, rows0, rows1,
              acc, dega, sem_g, sem_pq, sem_s, sem_d):
    c = lax.axis_index("c")
    s = lax.axis_index("s")
    wid = c * NSUB + s
    base = s * ROWS_PER_SUB

    zero16 = jnp.zeros((LANES,), jnp.float32)

    def zrow(e, carry):
        for d8 in range(D // LANES):
            rows0[e, pl.ds(d8 * LANES, LANES)] = zero16
        return carry

    lax.fori_loop(0, CHUNK, zrow, 0)

    for k in range(ROWS_PER_SUB // CHUNK):
        pltpu.sync_copy(rows0, acc.at[pl.ds(base + k * CHUNK, CHUNK)])
        pltpu.sync_copy(rows0.at[0],
                        dega.at[pl.ds(base + k * CHUNK, CHUNK)])

    pltpu.sync_copy(b_hbm, b_v)
    plsc.subcore_barrier()

    bb = b_v[...]
    bufs = (rows0, rows1)
    dstrows = (dstrow0, dstrow1)

    def start_gather(j, buf):
        jb = pl.multiple_of(j * CHUNK, CHUNK)
        pltpu.async_copy(feat_hbm.at[scmp.at[pl.ds(jb, CHUNK)]],
                         buf, sem_g)

    def wait_gather(buf):
        pltpu.make_async_copy(feat_hbm.at[pl.ds(0, CHUNK)], buf, sem_g).wait()

    def wait_scatter(buf):
        pltpu.make_async_copy(feat_hbm.at[pl.ds(0, CHUNK)], buf, sem_s).wait()

    def sblock(b, pend):
        pltpu.sync_copy(src_hbm.at[wid, b], src_v)
        pltpu.sync_copy(dst_hbm.at[wid, b], dst_v)

        pq = [pltpu.async_copy(p_hbm.at[src_v.at[js]], r_v.at[js], sem_pq)
              for js in range(SB)]
        pq += [pltpu.async_copy(q_hbm.at[dst_v.at[js]], qd_v.at[js], sem_pq)
               for js in range(SB)]
        for dsc in pq:
            dsc.wait()

        def cstep(j, off1):
            def ck(k8, off2):
                sl = pl.ds(k8 * LANES, LANES)
                s16 = src_v[j, sl]
                d16 = dst_v[j, sl]
                rr = jnp.maximum(r_v[j, sl] + qd_v[j, sl] + bb, 0.0)
                rr = jnp.where(s16 != d16, rr, 0.0)
                r_v[j, sl] = rr
                m = rr > 0.0
                plsc.store_compressed(scmp.at[pl.ds(off2, LANES)], s16,
                                      mask=m)
                plsc.store_compressed(dcmp.at[pl.ds(off2, LANES)], d16,
                                      mask=m)
                plsc.store_compressed(rcmp.at[pl.ds(off2, LANES)], rr,
                                      mask=m)
                return off2 + jnp.sum(m.astype(jnp.int32))
            return lax.fori_loop(0, CHUNK // LANES, ck, off1)

        off = lax.fori_loop(0, SB, cstep, jnp.int32(0))
        nact = (off + CHUNK - 1) // CHUNK

        for js in range(SB):
            pltpu.async_copy(r_v.at[js], dega.at[src_v.at[js]], sem_d,
                             add=True)

        iota16 = lax.iota(jnp.int32, LANES)

        @pl.when(nact > 0)
        def _pad_tail():
            lastb = (nact - 1) * CHUNK
            offv = jnp.full((LANES,), off, jnp.int32)
            for k8 in range(CHUNK // LANES):
                pos0 = lastb + k8 * LANES
                posv = jnp.full((LANES,), pos0, jnp.int32) + iota16
                mpad = posv >= offv
                dummy = (jnp.full((LANES,), N_NODES + k8 * LANES,
                                  jnp.int32) + iota16)
                sl = pl.ds(pos0, LANES)
                scmp[sl] = jnp.where(mpad, dummy, scmp[sl])
                dcmp[sl] = jnp.where(mpad, dummy, dcmp[sl])
                rcmp[sl] = jnp.where(mpad, 0.0, rcmp[sl])

        @pl.when((nact > 0) & (pend == 1))
        def _drain_xblock():
            wait_scatter(rows0)

        @pl.when(nact > 0)
        def _g0():
            start_gather(0, rows0)

        def mpair(jj, carry1):
            for par in range(2):
                j = 2 * jj + par
                buf = bufs[par]
                other = bufs[1 - par]
                dstrow = dstrows[par]

                @pl.when(j < nact)
                def _do_chunk():
                    wait_gather(buf)

                    @pl.when(j >= 1)
                    def _drain_prev():
                        wait_scatter(other)

                    @pl.when(j + 1 < nact)
                    def _gn():
                        start_gather(j + 1, other)

                    for k8 in range(CHUNK // LANES):
                        sl = pl.ds(k8 * LANES, LANES)
                        dsrc = pl.multiple_of(j * CHUNK + k8 * LANES, 8)
                        dstrow[0, sl] = dcmp[pl.ds(dsrc, LANES)]

                    @plsc.parallel_loop(0, CHUNK, unroll=4)
                    def _escale(e):
                        ee = jnp.full((LANES,), j * CHUNK + e, jnp.int32)
                        rv = plsc.load_gather(rcmp, [ee])
                        for d8 in range(D // LANES):
                            sl = pl.ds(d8 * LANES, LANES)
                            buf[e, sl] = buf[e, sl] * rv

                    pltpu.async_copy(buf, acc.at[dstrow.at[0]], sem_s,
                                     add=True)

            return carry1

        lax.fori_loop(0, SB // 2, mpair, 0)

        pltpu.make_async_copy(feat_hbm.at[pl.ds(0, SB)], r_v, sem_d).wait()
        return jnp.where(nact > 0, jnp.int32(1), pend)

    pend_f = lax.fori_loop(0, NSB, sblock, jnp.int32(0))

    @pl.when(pend_f == 1)
    def _drain_last():
        wait_scatter(rows0)

    plsc.subcore_barrier()

    pltpu.sync_copy(acc.at[pl.ds(base, ROWS_PER_SUB)],
                    adj_out.at[c].at[pl.ds(base, ROWS_PER_SUB)])
    pltpu.sync_copy(dega.at[pl.ds(base, ROWS_PER_SUB)],
                    deg_out.at[c].at[pl.ds(base, ROWS_PER_SUB)])


def kernel(x, edge_index, W_vel, b_vel, W_res, b_res, W_lap, b_lap):
    n, d = x.shape
    e = edge_index.shape[1]

    xp = jnp.zeros((NPAD, D), jnp.float32).at[:n].set(x)
    pad_ids = jnp.arange(EPAD - e, dtype=jnp.int32) % n
    srcp = jnp.concatenate([edge_index[0], pad_ids])
    dstp = jnp.concatenate([edge_index[1], pad_ids])
    srcp = srcp.reshape(NCHUNK, NTILES, CHUNK).transpose(1, 0, 2)
    dstp = dstp.reshape(NCHUNK, NTILES, CHUNK).transpose(1, 0, 2)
    srcp = srcp.reshape(NTILES, NSB, SB, CHUNK)
    dstp = dstp.reshape(NTILES, NSB, SB, CHUNK)
    w1 = W_res[0, :d]
    w2 = W_res[0, d:]
    b16 = jnp.full((LANES,), b_res[0], jnp.float32)

    wl_t = W_lap.T
    v, feat, p, q = _tc_prep1(xp, W_vel.T, b_vel, wl_t, b_lap, w1, w2)
    for it in range(ITERS):
        adj, deg = _sc_edges(feat, p.reshape(NPAD), q.reshape(NPAD),
                             srcp, dstp, b16)
        d0 = deg[0].reshape(NPAD, 1)
        d1 = deg[1].reshape(NPAD, 1)
        if it < ITERS - 1:
            xp, v, feat, p, q = _tc_upprep(xp, v, feat, adj[0], adj[1],
                                           d0, d1, wl_t, b_lap, w1, w2)
        else:
            xp = _tc_update(xp, v, feat, adj[0], adj[1], d0, d1)
    return xp[:n]
